# baseline (speedup 1.0000x reference)
import functools

import jax
import jax.numpy as jnp
from jax.experimental import pallas as pl

_B = 4
_N = 10000
_NT = 4 * 10000
_NF = 2
_WIN = 16
_NH = 64
_E = 160000
_LH = 40

_KB = 6400


def _fc1_body(x_ref, w_ref, b_ref, o_ref, *, ksteps):
    k = pl.program_id(0)

    @pl.when(k == 0)
    def _init():
        o_ref[...] = jnp.zeros_like(o_ref)

    o_ref[...] += jax.lax.dot_general(
        x_ref[...], w_ref[...], (((1,), (1,)), ((), ())),
        preferred_element_type=jnp.float32)

    @pl.when(k == ksteps - 1)
    def _fin():
        o_ref[...] = jnp.maximum(o_ref[...] + b_ref[...], 0.0)


def _fc1(xrows, fc1_w, fc1_b):
    R, K = xrows.shape
    ksteps = K // _KB
    body = functools.partial(_fc1_body, ksteps=ksteps)
    return pl.pallas_call(
        body,
        grid=(ksteps,),
        in_specs=[
            pl.BlockSpec((R, _KB), lambda k: (0, k)),
            pl.BlockSpec((_NH, _KB), lambda k: (0, k)),
            pl.BlockSpec((1, _NH), lambda k: (0, 0)),
        ],
        out_specs=pl.BlockSpec((R, _NH), lambda k: (0, 0)),
        out_shape=jax.ShapeDtypeStruct((R, _NH), jnp.float32),
    )(xrows, fc1_w, fc1_b.reshape(1, _NH))


def _aggregate(xs, row, col, ew):
    offs = (jnp.arange(_WIN, dtype=jnp.int32) * _N)[:, None]
    gidx = (row[None, :] + offs).reshape(-1)
    sidx = (col[None, :] + offs).reshape(-1)
    flat = xs.reshape(_WIN * _N, _NH)
    vals = flat[gidx] * jnp.tile(ew, _WIN)[:, None]
    agg = jax.ops.segment_sum(vals, sidx, num_segments=_WIN * _N)
    return agg.reshape(_WIN, _N, _NH)


def _gcn_layer(X, w, b, row, col, ew, dinv):
    XW = jax.lax.dot_general(X, w, (((2,), (1,)), ((), ())),
                             preferred_element_type=jnp.float32)
    xs = XW[:, :_N, :] * dinv[None, :, None]
    agg = _aggregate(xs, row, col, ew) + xs
    low = agg * dinv[None, :, None] + b
    high = XW[:, _N:, :] + b
    return jnp.concatenate([low, high], axis=1)


def _bn_t(x, g, be):
    m = x.mean(axis=1, keepdims=True)
    v = x.var(axis=1, keepdims=True)
    return (x - m) / jnp.sqrt(v + 1e-5) * g + be


def _run_lstm(seq, wih, whh, bih, bhh):
    def step(carry, xt):
        h, c = carry
        gates = xt @ wih.T + h @ whh.T + bih + bhh
        i, f, gg, o = jnp.split(gates, 4, axis=-1)
        i = jax.nn.sigmoid(i)
        f = jax.nn.sigmoid(f)
        gg = jnp.tanh(gg)
        o = jax.nn.sigmoid(o)
        c = f * c + i * gg
        h = o * jnp.tanh(c)
        return (h, c), h
    h0 = jnp.zeros((seq.shape[1], _LH), dtype=seq.dtype)
    (_, _), hs = jax.lax.scan(step, (h0, h0), seq)
    return hs


def kernel(data, edge_index, edge_attr, conv_w, conv_b, gcn1_w, gcn1_b,
           gcn2_w, gcn2_b, bn1_g, bn1_b, bn2_g, bn2_b, fc1_w, fc1_b,
           wih_f, whh_f, bih_f, bhh_f, wih_b, whh_b, bih_b, bhh_b,
           cls_w1, cls_b1, cls_w2, cls_b2):
    x0 = jnp.einsum('nfw,of->wno', data, conv_w) + conv_b[None, None, :]

    row = edge_index[0]
    col = edge_index[1]
    ew = edge_attr.reshape(-1)

    deg = jax.ops.segment_sum(ew, col, num_segments=_N) + 1.0
    dinv = jax.lax.rsqrt(deg)

    x1 = _bn_t(jax.nn.relu(_gcn_layer(x0, gcn1_w, gcn1_b, row, col, ew, dinv)),
               bn1_g, bn1_b) + x0
    x2 = _bn_t(jax.nn.relu(_gcn_layer(x1, gcn2_w, gcn2_b, row, col, ew, dinv)),
               bn2_g, bn2_b) + x1

    xc = x2.reshape(_WIN * _B, _N * _NH)
    X = _fc1(xc, fc1_w, fc1_b).reshape(_WIN, _B, _NH)
    hs_f = _run_lstm(X, wih_f, whh_f, bih_f, bhh_f)
    hs_b = _run_lstm(X[::-1], wih_b, whh_b, bih_b, bhh_b)[::-1]
    r_out = jnp.concatenate([hs_f, hs_b], axis=-1)
    x_step = r_out[-1]
    hcl = jax.nn.relu(x_step @ cls_w1.T + cls_b1)
    dec_score = hcl @ cls_w2.T + cls_b2
    return dec_score


# speedup vs baseline: 3.7432x; 3.7432x over previous
import functools

import jax
import jax.numpy as jnp
from jax.experimental import pallas as pl

_B = 4
_N = 10000
_NT = 4 * 10000
_NF = 2
_WIN = 16
_NH = 64
_E = 160000
_LH = 40

_KB = 6400


def _fc1_body(x_ref, w_ref, b_ref, o_ref, *, ksteps):
    k = pl.program_id(0)

    @pl.when(k == 0)
    def _init():
        o_ref[...] = jnp.zeros_like(o_ref)

    o_ref[...] += jax.lax.dot_general(
        x_ref[...], w_ref[...], (((1,), (1,)), ((), ())),
        preferred_element_type=jnp.float32)

    @pl.when(k == ksteps - 1)
    def _fin():
        o_ref[...] = jnp.maximum(o_ref[...] + b_ref[...], 0.0)


def _fc1(xrows, fc1_w, fc1_b):
    R, K = xrows.shape
    ksteps = K // _KB
    body = functools.partial(_fc1_body, ksteps=ksteps)
    return pl.pallas_call(
        body,
        grid=(ksteps,),
        in_specs=[
            pl.BlockSpec((R, _KB), lambda k: (0, k)),
            pl.BlockSpec((_NH, _KB), lambda k: (0, k)),
            pl.BlockSpec((1, _NH), lambda k: (0, 0)),
        ],
        out_specs=pl.BlockSpec((R, _NH), lambda k: (0, 0)),
        out_shape=jax.ShapeDtypeStruct((R, _NH), jnp.float32),
    )(xrows, fc1_w, fc1_b.reshape(1, _NH))


def _gcn(x, w, b, row, col, norm, dinv2):
    xw = x @ w.T
    agg = jax.ops.segment_sum(norm[:, None] * xw[row], col,
                              num_segments=_NT)
    return agg + dinv2[:, None] * xw + b


def _bn(x, g, be):
    m = x.mean(0)
    v = x.var(0)
    return (x - m) / jnp.sqrt(v + 1e-5) * g + be


def _run_lstm(seq, wih, whh, bih, bhh):
    def step(carry, xt):
        h, c = carry
        gates = xt @ wih.T + h @ whh.T + bih + bhh
        i, f, gg, o = jnp.split(gates, 4, axis=-1)
        i = jax.nn.sigmoid(i)
        f = jax.nn.sigmoid(f)
        gg = jnp.tanh(gg)
        o = jax.nn.sigmoid(o)
        c = f * c + i * gg
        h = o * jnp.tanh(c)
        return (h, c), h
    h0 = jnp.zeros((seq.shape[1], _LH), dtype=seq.dtype)
    (_, _), hs = jax.lax.scan(step, (h0, h0), seq)
    return hs


def kernel(data, edge_index, edge_attr, conv_w, conv_b, gcn1_w, gcn1_b,
           gcn2_w, gcn2_b, bn1_g, bn1_b, bn2_g, bn2_b, fc1_w, fc1_b,
           wih_f, whh_f, bih_f, bhh_f, wih_b, whh_b, bih_b, bhh_b,
           cls_w1, cls_b1, cls_w2, cls_b2):
    x = data.reshape(-1, _N, _NF, _WIN).transpose(0, 2, 1, 3)
    x = jnp.einsum('bcnw,oc->bonw', x, conv_w) + conv_b[None, :, None, None]
    x = x.transpose(0, 2, 3, 1).reshape(-1, _WIN, _NH)

    row = edge_index[0]
    col = edge_index[1]
    ew = edge_attr.reshape(-1)

    deg_small = jax.ops.segment_sum(ew, col, num_segments=_N) + 1.0
    dinv_small = jax.lax.rsqrt(deg_small)
    dinv = jnp.concatenate(
        [dinv_small, jnp.ones((_NT - _N,), dtype=dinv_small.dtype)])
    norm = dinv[row] * ew * dinv[col]
    dinv2 = dinv * dinv

    outs = []
    for l in range(_WIN):
        h = x[:, l, :]
        x1 = _bn(jax.nn.relu(_gcn(h, gcn1_w, gcn1_b, row, col, norm, dinv2)),
                 bn1_g, bn1_b) + h
        x2 = _bn(jax.nn.relu(_gcn(x1, gcn2_w, gcn2_b, row, col, norm, dinv2)),
                 bn2_g, bn2_b) + x1
        outs.append(x2)
    xc = jnp.stack(outs, 0).reshape(_WIN * _B, _N * _NH)
    X = _fc1(xc, fc1_w, fc1_b).reshape(_WIN, _B, _NH)
    hs_f = _run_lstm(X, wih_f, whh_f, bih_f, bhh_f)
    hs_b = _run_lstm(X[::-1], wih_b, whh_b, bih_b, bhh_b)[::-1]
    r_out = jnp.concatenate([hs_f, hs_b], axis=-1)
    x_step = r_out[-1]
    hcl = jax.nn.relu(x_step @ cls_w1.T + cls_b1)
    dec_score = hcl @ cls_w2.T + cls_b2
    return dec_score
